# baseline (speedup 1.0000x reference)
import functools

import jax
import jax.numpy as jnp
from jax import lax
from jax.experimental import pallas as pl
from jax.experimental.pallas import tpu as pltpu
from jax.experimental.pallas import tpu_sc as plsc

N = 8192
IN_CHS = 4096
RED = 1024
K = 2048
M_BLK = 256
GRID_M = N // M_BLK

NCORES = 2
NSUB = 16
NWORKERS = NCORES * NSUB
ROWS_PER_W = N // NWORKERS
NVREG_ALL = N // 16
NVREG_OWN = ROWS_PER_W // 16
CHUNK = 16


def _scores_body(x_ref, w1_ref, b1_ref, w2_ref, b2_ref, g_ref, s_ref, o_ref):
    i = pl.program_id(0)
    xb = x_ref[...]
    o_ref[...] = xb
    h = lax.dot_general(
        xb, w1_ref[...], (((1,), (1,)), ((), ())),
        preferred_element_type=jnp.float32,
    )
    h = jnp.maximum(h + b1_ref[...], 0.0)
    z = jnp.dot(h, w2_ref[...], preferred_element_type=jnp.float32)
    z = z + b2_ref[0, 0]
    a = 1.0 / (1.0 + jnp.exp(-z))
    s_ref[pl.ds(i * M_BLK, M_BLK), :] = a + g_ref[pl.ds(i * M_BLK, M_BLK), :]


def _sc_gate_body(scores_hbm, out_hbm, zeros_hbm, mask_hbm,
                  sc_v, keys_v, hist_v, mask_v, list_v, zrows_v, sem):
    wid = lax.axis_index("s") * NCORES + lax.axis_index("c")
    base = wid * ROWS_PER_W
    pltpu.sync_copy(scores_hbm, sc_v)
    pltpu.sync_copy(zeros_hbm, zrows_v)

    zeros16 = jnp.zeros((16,), jnp.int32)
    ones16 = jnp.ones((16,), jnp.int32)
    i16 = lax.iota(jnp.int32, 16)

    p = jnp.uint32(0)
    pm = jnp.uint32(0)
    k_rem = jnp.int32(K)
    for shift in (24, 16, 8, 0):
        sh = jnp.uint32(shift)
        for j in range(16):
            hist_v[pl.ds(16 * j, 16)] = zeros16

        if shift == 24:
            def hist_step(j, carry, sh=sh):
                for u_ in range(4):
                    off = 64 * j + 16 * u_
                    u = plsc.bitcast(sc_v[pl.ds(off, 16)], jnp.uint32)
                    flip = jnp.where(
                        u >= jnp.uint32(0x80000000),
                        jnp.uint32(0xFFFFFFFF),
                        jnp.uint32(0x80000000),
                    )
                    k16 = u ^ flip
                    keys_v[pl.ds(off, 16)] = k16
                    byte = (k16 >> sh).astype(jnp.int32)
                    plsc.addupdate_scatter(hist_v, [byte], ones16)
                return carry
        else:
            def hist_step(j, carry, pm=pm, p=p, sh=sh):
                for u_ in range(4):
                    k16 = keys_v[pl.ds(64 * j + 16 * u_, 16)]
                    part = (k16 & pm) == p
                    byte = ((k16 >> sh) & jnp.uint32(0xFF)).astype(jnp.int32)
                    plsc.addupdate_scatter(hist_v, [byte], ones16, mask=part)
                return carry

        lax.fori_loop(0, NVREG_ALL // 4, hist_step, jnp.int32(0))

        running = jnp.int32(0)
        bpop = jnp.int32(0)
        gh = []
        for j in range(15, -1, -1):
            g = hist_v[pl.ds(16 * j, 16)]
            gh.append(g)
            suf = lax.rev(plsc.cumsum(lax.rev(g, (0,))), (0,)) + running
            bpop = bpop + jnp.sum((suf >= k_rem).astype(jnp.int32))
            running = running + jnp.sum(g)
        b = bpop - jnp.int32(1)
        cgt = jnp.int32(0)
        for jj, g in enumerate(gh):
            idxs = i16 + 16 * (15 - jj)
            cgt = cgt + jnp.sum(jnp.where(idxs > b, g, 0))
        k_rem = k_rem - cgt
        p = p | (b.astype(jnp.uint32) << sh)
        pm = pm | (jnp.uint32(0xFF) << sh)

    t = p
    need = k_rem

    def eq_pre(j, acc):
        k16 = keys_v[pl.ds(16 * j, 16)]
        return acc + jnp.sum((k16 == t).astype(jnp.int32))

    prefix_eq = lax.fori_loop(0, wid * NVREG_OWN, eq_pre, jnp.int32(0))

    run_eq = prefix_eq
    off = jnp.int32(0)
    for j in range(NVREG_OWN):
        k16 = keys_v[pl.ds(base + 16 * j, 16)]
        gt = k16 > t
        eq = k16 == t
        eq_i = eq.astype(jnp.int32)
        rank = run_eq + plsc.cumsum(eq_i)
        sel = jnp.logical_or(gt, jnp.logical_and(eq, rank <= need))
        mask_v[pl.ds(16 * j, 16)] = sel.astype(jnp.float32)
        run_eq = run_eq + jnp.sum(eq_i)
        unsel = jnp.logical_not(sel)
        un_i = unsel.astype(jnp.int32)
        pos = off + plsc.cumsum(un_i) - 1
        rowid = base + 16 * j + i16
        plsc.store_scatter(list_v, [pos], rowid, mask=unsel)
        off = off + jnp.sum(un_i)

    pltpu.sync_copy(mask_v, mask_hbm.at[pl.ds(base, ROWS_PER_W)])

    nchunks = (off + (CHUNK - 1)) // CHUNK

    def fire_step(j, carry):
        vec = list_v[pl.ds(CHUNK * j, CHUNK)]
        valid = (CHUNK * j + i16) < off
        lane0 = jnp.sum(jnp.where(i16 == 0, vec, 0))
        fixed = jnp.where(valid, vec, lane0)
        pltpu.async_copy(zrows_v, out_hbm.at[fixed], sem)
        return carry

    lax.fori_loop(0, nchunks, fire_step, jnp.int32(0))

    def drain_step(j, carry):
        pltpu.make_async_copy(zeros_hbm, zrows_v, sem).wait()
        return carry

    lax.fori_loop(0, nchunks, drain_step, jnp.int32(0))


@functools.partial(
    pl.kernel,
    out_type=jax.ShapeDtypeStruct((N,), jnp.float32),
    mesh=plsc.VectorSubcoreMesh(
        core_axis_name="c", subcore_axis_name="s", num_cores=NCORES
    ),
    compiler_params=pltpu.CompilerParams(
        needs_layout_passes=False, has_side_effects=True
    ),
    scratch_types=[
        pltpu.VMEM((N,), jnp.float32),
        pltpu.VMEM((N,), jnp.uint32),
        pltpu.VMEM((256,), jnp.int32),
        pltpu.VMEM((ROWS_PER_W,), jnp.float32),
        pltpu.VMEM((ROWS_PER_W,), jnp.int32),
        pltpu.VMEM((CHUNK, IN_CHS), jnp.float32),
        pltpu.SemaphoreType.DMA,
    ],
)
def _sc_gate(scores_hbm, out_hbm, zeros_hbm, mask_hbm, *scratch):
    _sc_gate_body(scores_hbm, out_hbm, zeros_hbm, mask_hbm, *scratch)


@jax.jit
def kernel(x, W1, b1, W2, b2, gumbels):
    b1r = b1.reshape(1, RED)
    w2c = W2.reshape(RED, 1)
    b2r = b2.reshape(1, 1)
    g2 = gumbels.reshape(N, 1)

    scores, out = pl.pallas_call(
        _scores_body,
        grid=(GRID_M,),
        in_specs=[
            pl.BlockSpec((M_BLK, IN_CHS), lambda i: (i, 0)),
            pl.BlockSpec((RED, IN_CHS), lambda i: (0, 0)),
            pl.BlockSpec((1, RED), lambda i: (0, 0)),
            pl.BlockSpec((RED, 1), lambda i: (0, 0)),
            pl.BlockSpec((1, 1), lambda i: (0, 0)),
            pl.BlockSpec((N, 1), lambda i: (0, 0)),
        ],
        out_specs=[
            pl.BlockSpec((N, 1), lambda i: (0, 0)),
            pl.BlockSpec((M_BLK, IN_CHS), lambda i: (i, 0)),
        ],
        out_shape=[
            jax.ShapeDtypeStruct((N, 1), jnp.float32),
            jax.ShapeDtypeStruct((N, IN_CHS), jnp.float32),
        ],
    )(x, W1, b1r, w2c, b2r, g2)

    zrows = jnp.zeros((CHUNK, IN_CHS), jnp.float32)
    mask = _sc_gate(scores.reshape(N), out, zrows)
    out, _ = lax.optimization_barrier((out, mask))
    return out


# speedup vs baseline: 1.0937x; 1.0937x over previous
import functools

import jax
import jax.numpy as jnp
from jax import lax
from jax.experimental import pallas as pl
from jax.experimental.pallas import tpu as pltpu
from jax.experimental.pallas import tpu_sc as plsc

N = 8192
IN_CHS = 4096
RED = 1024
K = 2048
M_BLK = 256
GRID_M = N // M_BLK

NCORES = 2
NSUB = 16
NWORKERS = NCORES * NSUB
ROWS_PER_W = N // NWORKERS
NVREG_OWN = ROWS_PER_W // 16
CHUNK = 16


def _scores_body(x_ref, w1_ref, b1_ref, w2_ref, b2_ref, g_ref, s_ref, o_ref):
    i = pl.program_id(0)
    xb = x_ref[...]
    o_ref[...] = xb
    h = lax.dot_general(
        xb, w1_ref[...], (((1,), (1,)), ((), ())),
        preferred_element_type=jnp.float32,
    )
    h = jnp.maximum(h + b1_ref[...], 0.0)
    z = jnp.dot(h, w2_ref[...], preferred_element_type=jnp.float32)
    z = z + b2_ref[0, 0]
    a = 1.0 / (1.0 + jnp.exp(-z))
    s_ref[pl.ds(i * M_BLK, M_BLK), :] = a + g_ref[pl.ds(i * M_BLK, M_BLK), :]


def _mask_body(s_ref, m_ref):
    s = s_ref[...]
    u = lax.bitcast_convert_type(s, jnp.uint32)
    flip = jnp.where(
        u >= jnp.uint32(0x80000000),
        jnp.uint32(0xFFFFFFFF),
        jnp.uint32(0x80000000),
    )
    key = u ^ flip

    def bs(_, carry):
        lo, hi = carry
        d = hi - lo
        mid = lo + (d >> jnp.uint32(1)) + (d & jnp.uint32(1))
        cnt = jnp.sum((key >= mid).astype(jnp.int32))
        ok = cnt >= K
        return (jnp.where(ok, mid, lo), jnp.where(ok, hi, mid - jnp.uint32(1)))

    t, _ = lax.fori_loop(
        0, 32, bs, (jnp.uint32(0), jnp.uint32(0xFFFFFFFF))
    )

    gt = key > t
    eq = key == t
    need = (K - jnp.sum(gt.astype(jnp.int32))).astype(jnp.float32)

    eq_f = eq.astype(jnp.float32)
    r0 = lax.broadcasted_iota(jnp.int32, (128, 128), 0)
    c0 = lax.broadcasted_iota(jnp.int32, (128, 128), 1)
    upper_incl = (r0 <= c0).astype(jnp.float32)
    within = jnp.dot(eq_f, upper_incl, preferred_element_type=jnp.float32)
    row_tot = jnp.sum(eq_f, axis=1, keepdims=True)
    r1 = lax.broadcasted_iota(jnp.int32, (64, 64), 0)
    c1 = lax.broadcasted_iota(jnp.int32, (64, 64), 1)
    strict_lower = (c1 < r1).astype(jnp.float32)
    row_pref = jnp.dot(strict_lower, row_tot, preferred_element_type=jnp.float32)
    rank_incl = within + row_pref

    sel = jnp.logical_or(gt, jnp.logical_and(eq, rank_incl <= need))
    m_ref[...] = sel.astype(jnp.float32)


def _sc_scatter_body(mask_hbm, out_hbm, zeros_hbm, cnt_hbm,
                     mk_v, list_v, zrows_v, cnt_v, sem):
    wid = lax.axis_index("s") * NCORES + lax.axis_index("c")
    base = wid * ROWS_PER_W
    pltpu.sync_copy(mask_hbm.at[pl.ds(base, ROWS_PER_W)], mk_v)
    pltpu.sync_copy(zeros_hbm, zrows_v)

    i16 = lax.iota(jnp.int32, 16)

    off = jnp.int32(0)
    for j in range(NVREG_OWN):
        unsel = mk_v[pl.ds(16 * j, 16)] == 0.0
        un_i = unsel.astype(jnp.int32)
        pos = off + plsc.cumsum(un_i) - 1
        rowid = base + 16 * j + i16
        plsc.store_scatter(list_v, [pos], rowid, mask=unsel)
        off = off + jnp.sum(un_i)

    cnt_v[...] = jnp.zeros((16,), jnp.int32) + off
    pltpu.sync_copy(cnt_v, cnt_hbm.at[wid])

    nchunks = (off + (CHUNK - 1)) // CHUNK

    def fire_step(j, carry):
        vec = list_v[pl.ds(CHUNK * j, CHUNK)]
        valid = (CHUNK * j + i16) < off
        lane0 = jnp.sum(jnp.where(i16 == 0, vec, 0))
        fixed = jnp.where(valid, vec, lane0)
        pltpu.async_copy(zrows_v, out_hbm.at[fixed], sem)
        return carry

    lax.fori_loop(0, nchunks, fire_step, jnp.int32(0))

    def drain_step(j, carry):
        pltpu.make_async_copy(zeros_hbm, zrows_v, sem).wait()
        return carry

    lax.fori_loop(0, nchunks, drain_step, jnp.int32(0))


@functools.partial(
    pl.kernel,
    out_type=jax.ShapeDtypeStruct((NWORKERS, 16), jnp.int32),
    mesh=plsc.VectorSubcoreMesh(
        core_axis_name="c", subcore_axis_name="s", num_cores=NCORES
    ),
    compiler_params=pltpu.CompilerParams(
        needs_layout_passes=False, has_side_effects=True
    ),
    scratch_types=[
        pltpu.VMEM((ROWS_PER_W,), jnp.float32),
        pltpu.VMEM((ROWS_PER_W,), jnp.int32),
        pltpu.VMEM((CHUNK, IN_CHS), jnp.float32),
        pltpu.VMEM((16,), jnp.int32),
        pltpu.SemaphoreType.DMA,
    ],
)
def _sc_scatter(mask_hbm, out_hbm, zeros_hbm, cnt_hbm, *scratch):
    _sc_scatter_body(mask_hbm, out_hbm, zeros_hbm, cnt_hbm, *scratch)


@jax.jit
def kernel(x, W1, b1, W2, b2, gumbels):
    b1r = b1.reshape(1, RED)
    w2c = W2.reshape(RED, 1)
    b2r = b2.reshape(1, 1)
    g2 = gumbels.reshape(N, 1)

    scores, out = pl.pallas_call(
        _scores_body,
        grid=(GRID_M,),
        in_specs=[
            pl.BlockSpec((M_BLK, IN_CHS), lambda i: (i, 0)),
            pl.BlockSpec((RED, IN_CHS), lambda i: (0, 0)),
            pl.BlockSpec((1, RED), lambda i: (0, 0)),
            pl.BlockSpec((RED, 1), lambda i: (0, 0)),
            pl.BlockSpec((1, 1), lambda i: (0, 0)),
            pl.BlockSpec((N, 1), lambda i: (0, 0)),
        ],
        out_specs=[
            pl.BlockSpec((N, 1), lambda i: (0, 0)),
            pl.BlockSpec((M_BLK, IN_CHS), lambda i: (i, 0)),
        ],
        out_shape=[
            jax.ShapeDtypeStruct((N, 1), jnp.float32),
            jax.ShapeDtypeStruct((N, IN_CHS), jnp.float32),
        ],
    )(x, W1, b1r, w2c, b2r, g2)

    mask = pl.pallas_call(
        _mask_body,
        out_shape=jax.ShapeDtypeStruct((64, 128), jnp.float32),
    )(scores.reshape(64, 128))

    zrows = jnp.zeros((CHUNK, IN_CHS), jnp.float32)
    cnts = _sc_scatter(mask.reshape(N), out, zrows)
    out, _ = lax.optimization_barrier((out, cnts))
    return out
